# baseline (speedup 1.0000x reference)
import jax
import jax.numpy as jnp
from jax import lax
from jax.experimental import pallas as pl

N_DIM = 512
T = 25
NDIA = 32
DB = 8
RB = DB * T
GRID = NDIA // DB
NUM_L = 3
NUM_K = 4
ROWS = NDIA * T


def _body(l_ref, a_ref, v_ref, aux_ref, emb_ref, w1_ref, b1_ref, attrs_ref,
          hw_ref, hb_ref, gw_ref, gb_ref, out_ref):
    f32 = jnp.float32
    aux = aux_ref[...]

    def col(i):
        return aux[:, i:i + 1]

    def dot(x, w):
        return jax.lax.dot(x.astype(jnp.bfloat16), w.astype(jnp.bfloat16),
                           preferred_element_type=f32)

    t_i = lax.broadcasted_iota(jnp.int32, (RB, 1), 0) % T
    mask = (t_i.astype(f32) < col(0)).astype(f32)
    tmask = (t_i < (T - NUM_L)).astype(f32)

    sel = (col(1) >= col(2)).astype(f32)
    emb = emb_ref[...]
    embsel = sel * emb[0:1, :] + (1.0 - sel) * emb[1:2, :]

    Fl = l_ref[...] * mask + embsel
    Fa = a_ref[...] * mask
    Fv = v_ref[...] * mask

    b1 = b1_ref[...]
    W1 = w1_ref[...]
    x1l = dot(Fl, W1) + b1
    x1a = dot(Fa, W1) + b1
    x1v = dot(Fv, W1) + b1

    d_io = lax.broadcasted_iota(jnp.int32, (DB, RB), 0)
    r_io = lax.broadcasted_iota(jnp.int32, (DB, RB), 1)
    U = (r_io // T == d_io).astype(f32)
    d_io2 = lax.broadcasted_iota(jnp.int32, (RB, DB), 1)
    r_io2 = lax.broadcasted_iota(jnp.int32, (RB, DB), 0)
    UT = (r_io2 // T == d_io2).astype(f32)

    def gsum(x):
        return dot(UT, dot(U, x))

    Hl, Ha, Hv = x1l, x1a, x1v
    attrs = attrs_ref[...]
    inv_big = 1.0 / T
    inv_tri = 1.0 / 3.0
    for li in range(NUM_L):
        W = hw_ref[li]
        b = hb_ref[li:li + 1, :]
        ew12 = dot(attrs, W)
        ew1 = ew12[0:1, :]
        ew2 = ew12[1:2, :]
        xwl = dot(Hl, W)
        xwa = dot(Ha, W)
        xwv = dot(Hv, W)
        mbl = gsum(xwl * col(3)) * inv_big + ew1
        mba = gsum(xwa * col(4)) * inv_big + ew1
        mbv = gsum(xwv * col(5)) * inv_big + ew1
        ewtri = tmask * ew1 + (1.0 - tmask) * ew2
        mtri = (xwl * col(6) + xwa * col(7) + xwv * col(8)) * inv_tri + ewtri
        outs = []
        for g, mb in enumerate((mbl, mba, mbv)):
            wbig = col(9 + g)
            wtri = col(12)
            dd = wbig + wtri
            dinv = jnp.where(dd > 0, 1.0 / dd, 0.0)
            outs.append((wbig * mb + wtri * mtri) * dinv + b)
        Hl, Ha, Hv = outs

    Gl, Ga, Gv = x1l, x1a, x1v
    dinv_g = 1.0 / (T - 1 + 2)
    for kk in range(NUM_K):
        W = gw_ref[kk]
        b = gb_ref[kk:kk + 1, :]
        xpl = dot(Gl, W)
        xpa = dot(Ga, W)
        xpv = dot(Gv, W)
        T3 = xpl + xpa + xpv
        Gl = Gl + (gsum(xpl) + T3 - 2.0 * xpl) * dinv_g + b
        Ga = Ga + (gsum(xpa) + T3 - 2.0 * xpa) * dinv_g + b
        Gv = Gv + (gsum(xpv) + T3 - 2.0 * xpv) * dinv_g + b

    D = N_DIM
    for j, arr in enumerate((Fl, Hl, Gl, Fa, Ha, Ga, Fv, Hv, Gv)):
        out_ref[:, j * D:(j + 1) * D] = arr


def kernel(a, v, l, dia_len, qmask, epoch, speaker_emb, fc1_W, fc1_b,
           hyperedge_weight, EW_weight, hyperedge_attr1, hyperedge_attr2,
           hconv_W, hconv_b, gconv_W, gconv_b):
    f32 = jnp.float32
    num_edges = NDIA * (T + NUM_L)
    nnz = NDIA * (6 * T)

    dlen = jnp.broadcast_to(dia_len.astype(f32)[:, None], (NDIA, T)).reshape(ROWS)
    qmT = qmask.astype(f32).transpose(1, 0, 2).reshape(ROWS, 2)
    E = EW_weight[:nnz].astype(f32).reshape(NDIA, 6 * T)
    ewb = E[:, :3 * T].reshape(NDIA, 3, T)
    ewt = E[:, 3 * T:].reshape(NDIA, T, 3)
    hw = hyperedge_weight[:num_edges].astype(f32).reshape(NDIA, T + NUM_L)
    cols = [
        dlen,
        qmT[:, 0], qmT[:, 1],
        ewb[:, 0, :].reshape(ROWS),
        ewb[:, 1, :].reshape(ROWS),
        ewb[:, 2, :].reshape(ROWS),
        ewt[:, :, 0].reshape(ROWS),
        ewt[:, :, 1].reshape(ROWS),
        ewt[:, :, 2].reshape(ROWS),
        jnp.broadcast_to(hw[:, 0:1], (NDIA, T)).reshape(ROWS),
        jnp.broadcast_to(hw[:, 1:2], (NDIA, T)).reshape(ROWS),
        jnp.broadcast_to(hw[:, 2:3], (NDIA, T)).reshape(ROWS),
        hw[:, NUM_L:].reshape(ROWS),
    ]
    cols += [jnp.zeros((ROWS,), f32)] * 3
    aux = jnp.stack(cols, axis=1)

    attrs = jnp.stack([hyperedge_attr1, hyperedge_attr2], axis=0).astype(f32)
    b1 = fc1_b.reshape(1, N_DIM).astype(f32)

    row_spec = pl.BlockSpec((RB, N_DIM), lambda k: (k, 0))
    full2 = lambda arr: pl.BlockSpec(arr.shape, lambda k: (0,) * arr.ndim)

    out = pl.pallas_call(
        _body,
        grid=(GRID,),
        in_specs=[
            row_spec, row_spec, row_spec,
            pl.BlockSpec((RB, 16), lambda k: (k, 0)),
            full2(speaker_emb), full2(fc1_W), full2(b1), full2(attrs),
            full2(hconv_W), full2(hconv_b), full2(gconv_W), full2(gconv_b),
        ],
        out_specs=pl.BlockSpec((RB, 9 * N_DIM), lambda k: (k, 0)),
        out_shape=jax.ShapeDtypeStruct((ROWS, 9 * N_DIM), f32),
    )(l.astype(f32), a.astype(f32), v.astype(f32), aux,
      speaker_emb.astype(f32), fc1_W.astype(f32), b1, attrs,
      hconv_W.astype(f32), hconv_b.astype(f32),
      gconv_W.astype(f32), gconv_b.astype(f32))
    return out


# speedup vs baseline: 1.0188x; 1.0188x over previous
import jax
import jax.numpy as jnp
from jax import lax
from jax.experimental import pallas as pl

N_DIM = 512
T = 25
NDIA = 32
DB = 8
RB = DB * T
GRID = NDIA // DB
NUM_L = 3
NUM_K = 4
ROWS = NDIA * T


def _body(l_ref, a_ref, v_ref, aux_ref, emb_ref, w1_ref, b1_ref, attrs_ref,
          hw_ref, hb_ref, gw_ref, gb_ref, out_ref):
    f32 = jnp.float32
    aux = aux_ref[...]

    def col(i):
        return aux[:, i:i + 1]

    def dot(x, w):
        return jax.lax.dot(x, w, preferred_element_type=f32)

    t_i = lax.broadcasted_iota(jnp.int32, (RB, 1), 0) % T
    mask = (t_i.astype(f32) < col(0)).astype(f32)
    tmask = (t_i < (T - NUM_L)).astype(f32)

    sel = (col(1) >= col(2)).astype(f32)
    emb = emb_ref[...]
    embsel = sel * emb[0:1, :] + (1.0 - sel) * emb[1:2, :]

    Fl = l_ref[...] * mask + embsel
    Fa = a_ref[...] * mask
    Fv = v_ref[...] * mask

    b1 = b1_ref[...]
    W1 = w1_ref[...]
    x1l = dot(Fl, W1) + b1
    x1a = dot(Fa, W1) + b1
    x1v = dot(Fv, W1) + b1

    d_io = lax.broadcasted_iota(jnp.int32, (DB, RB), 0)
    r_io = lax.broadcasted_iota(jnp.int32, (DB, RB), 1)
    U = (r_io // T == d_io).astype(f32)
    d_io2 = lax.broadcasted_iota(jnp.int32, (RB, DB), 1)
    r_io2 = lax.broadcasted_iota(jnp.int32, (RB, DB), 0)
    UT = (r_io2 // T == d_io2).astype(f32)

    def gsum(x):
        return dot(UT, dot(U, x))

    Hl, Ha, Hv = x1l, x1a, x1v
    attrs = attrs_ref[...]
    inv_big = 1.0 / T
    inv_tri = 1.0 / 3.0
    for li in range(NUM_L):
        W = hw_ref[li]
        b = hb_ref[li:li + 1, :]
        ew12 = dot(attrs, W)
        ew1 = ew12[0:1, :]
        ew2 = ew12[1:2, :]
        xwl = dot(Hl, W)
        xwa = dot(Ha, W)
        xwv = dot(Hv, W)
        mbl = gsum(xwl * col(3)) * inv_big + ew1
        mba = gsum(xwa * col(4)) * inv_big + ew1
        mbv = gsum(xwv * col(5)) * inv_big + ew1
        ewtri = tmask * ew1 + (1.0 - tmask) * ew2
        mtri = (xwl * col(6) + xwa * col(7) + xwv * col(8)) * inv_tri + ewtri
        outs = []
        for g, mb in enumerate((mbl, mba, mbv)):
            wbig = col(9 + g)
            wtri = col(12)
            dd = wbig + wtri
            dinv = jnp.where(dd > 0, 1.0 / dd, 0.0)
            outs.append((wbig * mb + wtri * mtri) * dinv + b)
        Hl, Ha, Hv = outs

    Gl, Ga, Gv = x1l, x1a, x1v
    dinv_g = 1.0 / (T - 1 + 2)
    for kk in range(NUM_K):
        W = gw_ref[kk]
        b = gb_ref[kk:kk + 1, :]
        xpl = dot(Gl, W)
        xpa = dot(Ga, W)
        xpv = dot(Gv, W)
        T3 = xpl + xpa + xpv
        Gl = Gl + (gsum(xpl) + T3 - 2.0 * xpl) * dinv_g + b
        Ga = Ga + (gsum(xpa) + T3 - 2.0 * xpa) * dinv_g + b
        Gv = Gv + (gsum(xpv) + T3 - 2.0 * xpv) * dinv_g + b

    D = N_DIM
    for j, arr in enumerate((Fl, Hl, Gl, Fa, Ha, Ga, Fv, Hv, Gv)):
        out_ref[:, j * D:(j + 1) * D] = arr


def kernel(a, v, l, dia_len, qmask, epoch, speaker_emb, fc1_W, fc1_b,
           hyperedge_weight, EW_weight, hyperedge_attr1, hyperedge_attr2,
           hconv_W, hconv_b, gconv_W, gconv_b):
    f32 = jnp.float32
    num_edges = NDIA * (T + NUM_L)
    nnz = NDIA * (6 * T)

    dlen = jnp.broadcast_to(dia_len.astype(f32)[:, None], (NDIA, T)).reshape(ROWS)
    qmT = qmask.astype(f32).transpose(1, 0, 2).reshape(ROWS, 2)
    E = EW_weight[:nnz].astype(f32).reshape(NDIA, 6 * T)
    ewb = E[:, :3 * T].reshape(NDIA, 3, T)
    ewt = E[:, 3 * T:].reshape(NDIA, T, 3)
    hw = hyperedge_weight[:num_edges].astype(f32).reshape(NDIA, T + NUM_L)
    cols = [
        dlen,
        qmT[:, 0], qmT[:, 1],
        ewb[:, 0, :].reshape(ROWS),
        ewb[:, 1, :].reshape(ROWS),
        ewb[:, 2, :].reshape(ROWS),
        ewt[:, :, 0].reshape(ROWS),
        ewt[:, :, 1].reshape(ROWS),
        ewt[:, :, 2].reshape(ROWS),
        jnp.broadcast_to(hw[:, 0:1], (NDIA, T)).reshape(ROWS),
        jnp.broadcast_to(hw[:, 1:2], (NDIA, T)).reshape(ROWS),
        jnp.broadcast_to(hw[:, 2:3], (NDIA, T)).reshape(ROWS),
        hw[:, NUM_L:].reshape(ROWS),
    ]
    cols += [jnp.zeros((ROWS,), f32)] * 3
    aux = jnp.stack(cols, axis=1)

    attrs = jnp.stack([hyperedge_attr1, hyperedge_attr2], axis=0).astype(f32)
    b1 = fc1_b.reshape(1, N_DIM).astype(f32)

    row_spec = pl.BlockSpec((RB, N_DIM), lambda k: (k, 0))
    full2 = lambda arr: pl.BlockSpec(arr.shape, lambda k: (0,) * arr.ndim)

    out = pl.pallas_call(
        _body,
        grid=(GRID,),
        in_specs=[
            row_spec, row_spec, row_spec,
            pl.BlockSpec((RB, 16), lambda k: (k, 0)),
            full2(speaker_emb), full2(fc1_W), full2(b1), full2(attrs),
            full2(hconv_W), full2(hconv_b), full2(gconv_W), full2(gconv_b),
        ],
        out_specs=pl.BlockSpec((RB, 9 * N_DIM), lambda k: (k, 0)),
        out_shape=jax.ShapeDtypeStruct((ROWS, 9 * N_DIM), f32),
    )(l.astype(f32), a.astype(f32), v.astype(f32), aux,
      speaker_emb.astype(f32), fc1_W.astype(f32), b1, attrs,
      hconv_W.astype(f32), hconv_b.astype(f32),
      gconv_W.astype(f32), gconv_b.astype(f32))
    return out


# speedup vs baseline: 1.1233x; 1.1026x over previous
import jax
import jax.numpy as jnp
from jax import lax
from jax.experimental import pallas as pl

N_DIM = 512
T = 25
NDIA = 32
DB = 8
RB = DB * T
GRID = NDIA // DB
NUM_L = 3
NUM_K = 4
ROWS = NDIA * T


def _body(l_ref, a_ref, v_ref, aux_ref, emb_ref, w1_ref, b1_ref, attrs_ref,
          hw_ref, hb_ref, gw_ref, gb_ref, out_ref):
    f32 = jnp.float32
    aux = aux_ref[...]
    SR = 3 * RB

    def col(i):
        return aux[:, i:i + 1]

    def scol(i, j, k):
        return jnp.concatenate([col(i), col(j), col(k)], axis=0)

    def dot(x, w):
        return jax.lax.dot(x, w, preferred_element_type=f32)

    t_i = lax.broadcasted_iota(jnp.int32, (RB, 1), 0) % T
    mask = (t_i.astype(f32) < col(0)).astype(f32)
    tmask = (t_i < (T - NUM_L)).astype(f32)

    sel = (col(1) >= col(2)).astype(f32)
    emb = emb_ref[...]
    embsel = sel * emb[0:1, :] + (1.0 - sel) * emb[1:2, :]

    F = jnp.concatenate([l_ref[...] * mask + embsel,
                         a_ref[...] * mask,
                         v_ref[...] * mask], axis=0)
    x1 = dot(F, w1_ref[...]) + b1_ref[...]

    g_io = lax.broadcasted_iota(jnp.int32, (3 * DB, SR), 0)
    r_io = lax.broadcasted_iota(jnp.int32, (3 * DB, SR), 1)
    U = (r_io // T == g_io).astype(f32)
    g_io2 = lax.broadcasted_iota(jnp.int32, (SR, 3 * DB), 1)
    r_io2 = lax.broadcasted_iota(jnp.int32, (SR, 3 * DB), 0)
    UT = (r_io2 // T == g_io2).astype(f32)

    def gsum(x):
        return dot(UT, dot(U, x))

    def tile3(x):
        return jnp.concatenate([x, x, x], axis=0)

    ewb_s = scol(3, 4, 5)
    wbig_s = scol(9, 10, 11)
    wtri_s = tile3(col(12))
    dd = wbig_s + wtri_s
    dinv_s = jnp.where(dd > 0, 1.0 / dd, 0.0)

    H = x1
    attrs = attrs_ref[...]
    inv_big = 1.0 / T
    inv_tri = 1.0 / 3.0
    for li in range(NUM_L):
        W = hw_ref[li]
        b = hb_ref[li:li + 1, :]
        ew12 = dot(attrs, W)
        ew1 = ew12[0:1, :]
        ew2 = ew12[1:2, :]
        xw = dot(H, W)
        mb = gsum(xw * ewb_s) * inv_big + ew1
        ewtri = tmask * ew1 + (1.0 - tmask) * ew2
        mtri = (xw[0:RB] * col(6) + xw[RB:2 * RB] * col(7)
                + xw[2 * RB:] * col(8)) * inv_tri + ewtri
        H = (wbig_s * mb + wtri_s * tile3(mtri)) * dinv_s + b

    G = x1
    dinv_g = 1.0 / (T - 1 + 2)
    for kk in range(NUM_K):
        W = gw_ref[kk]
        b = gb_ref[kk:kk + 1, :]
        xp = dot(G, W)
        T3 = xp[0:RB] + xp[RB:2 * RB] + xp[2 * RB:]
        G = G + (gsum(xp) + tile3(T3) - 2.0 * xp) * dinv_g + b

    D = N_DIM
    for g in range(3):
        sl = slice(g * RB, (g + 1) * RB)
        out_ref[:, (3 * g) * D:(3 * g + 1) * D] = F[sl]
        out_ref[:, (3 * g + 1) * D:(3 * g + 2) * D] = H[sl]
        out_ref[:, (3 * g + 2) * D:(3 * g + 3) * D] = G[sl]


def kernel(a, v, l, dia_len, qmask, epoch, speaker_emb, fc1_W, fc1_b,
           hyperedge_weight, EW_weight, hyperedge_attr1, hyperedge_attr2,
           hconv_W, hconv_b, gconv_W, gconv_b):
    f32 = jnp.float32
    num_edges = NDIA * (T + NUM_L)
    nnz = NDIA * (6 * T)

    dlen = jnp.broadcast_to(dia_len.astype(f32)[:, None], (NDIA, T)).reshape(ROWS)
    qmT = qmask.astype(f32).transpose(1, 0, 2).reshape(ROWS, 2)
    E = EW_weight[:nnz].astype(f32).reshape(NDIA, 6 * T)
    ewb = E[:, :3 * T].reshape(NDIA, 3, T)
    ewt = E[:, 3 * T:].reshape(NDIA, T, 3)
    hw = hyperedge_weight[:num_edges].astype(f32).reshape(NDIA, T + NUM_L)
    cols = [
        dlen,
        qmT[:, 0], qmT[:, 1],
        ewb[:, 0, :].reshape(ROWS),
        ewb[:, 1, :].reshape(ROWS),
        ewb[:, 2, :].reshape(ROWS),
        ewt[:, :, 0].reshape(ROWS),
        ewt[:, :, 1].reshape(ROWS),
        ewt[:, :, 2].reshape(ROWS),
        jnp.broadcast_to(hw[:, 0:1], (NDIA, T)).reshape(ROWS),
        jnp.broadcast_to(hw[:, 1:2], (NDIA, T)).reshape(ROWS),
        jnp.broadcast_to(hw[:, 2:3], (NDIA, T)).reshape(ROWS),
        hw[:, NUM_L:].reshape(ROWS),
    ]
    cols += [jnp.zeros((ROWS,), f32)] * 3
    aux = jnp.stack(cols, axis=1)

    attrs = jnp.stack([hyperedge_attr1, hyperedge_attr2], axis=0).astype(f32)
    b1 = fc1_b.reshape(1, N_DIM).astype(f32)

    row_spec = pl.BlockSpec((RB, N_DIM), lambda k: (k, 0))
    full2 = lambda arr: pl.BlockSpec(arr.shape, lambda k: (0,) * arr.ndim)

    out = pl.pallas_call(
        _body,
        grid=(GRID,),
        in_specs=[
            row_spec, row_spec, row_spec,
            pl.BlockSpec((RB, 16), lambda k: (k, 0)),
            full2(speaker_emb), full2(fc1_W), full2(b1), full2(attrs),
            full2(hconv_W), full2(hconv_b), full2(gconv_W), full2(gconv_b),
        ],
        out_specs=pl.BlockSpec((RB, 9 * N_DIM), lambda k: (k, 0)),
        out_shape=jax.ShapeDtypeStruct((ROWS, 9 * N_DIM), f32),
    )(l.astype(f32), a.astype(f32), v.astype(f32), aux,
      speaker_emb.astype(f32), fc1_W.astype(f32), b1, attrs,
      hconv_W.astype(f32), hconv_b.astype(f32),
      gconv_W.astype(f32), gconv_b.astype(f32))
    return out


# speedup vs baseline: 1.1718x; 1.0432x over previous
import jax
import jax.numpy as jnp
from jax import lax
from jax.experimental import pallas as pl

N_DIM = 512
T = 25
NDIA = 32
DB = 16
RB = DB * T
GRID = NDIA // DB
NUM_L = 3
NUM_K = 4
ROWS = NDIA * T


def _body(l_ref, a_ref, v_ref, aux_ref, emb_ref, w1_ref, b1_ref, attrs_ref,
          hw_ref, hb_ref, gw_ref, gb_ref, out_ref):
    f32 = jnp.float32
    aux = aux_ref[...]
    SR = 3 * RB

    def col(i):
        return aux[:, i:i + 1]

    def scol(i, j, k):
        return jnp.concatenate([col(i), col(j), col(k)], axis=0)

    def dot(x, w):
        return jax.lax.dot(x, w, preferred_element_type=f32)

    t_i = lax.broadcasted_iota(jnp.int32, (RB, 1), 0) % T
    mask = (t_i.astype(f32) < col(0)).astype(f32)
    tmask = (t_i < (T - NUM_L)).astype(f32)

    sel = (col(1) >= col(2)).astype(f32)
    emb = emb_ref[...]
    embsel = sel * emb[0:1, :] + (1.0 - sel) * emb[1:2, :]

    F = jnp.concatenate([l_ref[...] * mask + embsel,
                         a_ref[...] * mask,
                         v_ref[...] * mask], axis=0)
    x1 = dot(F, w1_ref[...]) + b1_ref[...]

    g_io = lax.broadcasted_iota(jnp.int32, (3 * DB, SR), 0)
    r_io = lax.broadcasted_iota(jnp.int32, (3 * DB, SR), 1)
    U = (r_io // T == g_io).astype(f32)
    g_io2 = lax.broadcasted_iota(jnp.int32, (SR, 3 * DB), 1)
    r_io2 = lax.broadcasted_iota(jnp.int32, (SR, 3 * DB), 0)
    UT = (r_io2 // T == g_io2).astype(f32)

    def gsum(x):
        return dot(UT, dot(U, x))

    def tile3(x):
        return jnp.concatenate([x, x, x], axis=0)

    ewb_s = scol(3, 4, 5)
    wbig_s = scol(9, 10, 11)
    wtri_s = tile3(col(12))
    dd = wbig_s + wtri_s
    dinv_s = jnp.where(dd > 0, 1.0 / dd, 0.0)

    H = x1
    attrs = attrs_ref[...]
    inv_big = 1.0 / T
    inv_tri = 1.0 / 3.0
    for li in range(NUM_L):
        W = hw_ref[li]
        b = hb_ref[li:li + 1, :]
        ew12 = dot(attrs, W)
        ew1 = ew12[0:1, :]
        ew2 = ew12[1:2, :]
        xw = dot(H, W)
        mb = gsum(xw * ewb_s) * inv_big + ew1
        ewtri = tmask * ew1 + (1.0 - tmask) * ew2
        mtri = (xw[0:RB] * col(6) + xw[RB:2 * RB] * col(7)
                + xw[2 * RB:] * col(8)) * inv_tri + ewtri
        H = (wbig_s * mb + wtri_s * tile3(mtri)) * dinv_s + b

    G = x1
    dinv_g = 1.0 / (T - 1 + 2)
    for kk in range(NUM_K):
        W = gw_ref[kk]
        b = gb_ref[kk:kk + 1, :]
        xp = dot(G, W)
        T3 = xp[0:RB] + xp[RB:2 * RB] + xp[2 * RB:]
        G = G + (gsum(xp) + tile3(T3) - 2.0 * xp) * dinv_g + b

    D = N_DIM
    for g in range(3):
        sl = slice(g * RB, (g + 1) * RB)
        out_ref[:, (3 * g) * D:(3 * g + 1) * D] = F[sl]
        out_ref[:, (3 * g + 1) * D:(3 * g + 2) * D] = H[sl]
        out_ref[:, (3 * g + 2) * D:(3 * g + 3) * D] = G[sl]


def kernel(a, v, l, dia_len, qmask, epoch, speaker_emb, fc1_W, fc1_b,
           hyperedge_weight, EW_weight, hyperedge_attr1, hyperedge_attr2,
           hconv_W, hconv_b, gconv_W, gconv_b):
    f32 = jnp.float32
    num_edges = NDIA * (T + NUM_L)
    nnz = NDIA * (6 * T)

    dlen = jnp.broadcast_to(dia_len.astype(f32)[:, None], (NDIA, T)).reshape(ROWS)
    qmT = qmask.astype(f32).transpose(1, 0, 2).reshape(ROWS, 2)
    E = EW_weight[:nnz].astype(f32).reshape(NDIA, 6 * T)
    ewb = E[:, :3 * T].reshape(NDIA, 3, T)
    ewt = E[:, 3 * T:].reshape(NDIA, T, 3)
    hw = hyperedge_weight[:num_edges].astype(f32).reshape(NDIA, T + NUM_L)
    cols = [
        dlen,
        qmT[:, 0], qmT[:, 1],
        ewb[:, 0, :].reshape(ROWS),
        ewb[:, 1, :].reshape(ROWS),
        ewb[:, 2, :].reshape(ROWS),
        ewt[:, :, 0].reshape(ROWS),
        ewt[:, :, 1].reshape(ROWS),
        ewt[:, :, 2].reshape(ROWS),
        jnp.broadcast_to(hw[:, 0:1], (NDIA, T)).reshape(ROWS),
        jnp.broadcast_to(hw[:, 1:2], (NDIA, T)).reshape(ROWS),
        jnp.broadcast_to(hw[:, 2:3], (NDIA, T)).reshape(ROWS),
        hw[:, NUM_L:].reshape(ROWS),
    ]
    cols += [jnp.zeros((ROWS,), f32)] * 3
    aux = jnp.stack(cols, axis=1)

    attrs = jnp.stack([hyperedge_attr1, hyperedge_attr2], axis=0).astype(f32)
    b1 = fc1_b.reshape(1, N_DIM).astype(f32)

    row_spec = pl.BlockSpec((RB, N_DIM), lambda k: (k, 0))
    full2 = lambda arr: pl.BlockSpec(arr.shape, lambda k: (0,) * arr.ndim)

    out = pl.pallas_call(
        _body,
        grid=(GRID,),
        in_specs=[
            row_spec, row_spec, row_spec,
            pl.BlockSpec((RB, 16), lambda k: (k, 0)),
            full2(speaker_emb), full2(fc1_W), full2(b1), full2(attrs),
            full2(hconv_W), full2(hconv_b), full2(gconv_W), full2(gconv_b),
        ],
        out_specs=pl.BlockSpec((RB, 9 * N_DIM), lambda k: (k, 0)),
        out_shape=jax.ShapeDtypeStruct((ROWS, 9 * N_DIM), f32),
    )(l.astype(f32), a.astype(f32), v.astype(f32), aux,
      speaker_emb.astype(f32), fc1_W.astype(f32), b1, attrs,
      hconv_W.astype(f32), hconv_b.astype(f32),
      gconv_W.astype(f32), gconv_b.astype(f32))
    return out
